# baseline (speedup 1.0000x reference)
import functools

import jax
import jax.numpy as jnp
from jax import lax
from jax.experimental import pallas as pl
from jax.experimental.pallas import tpu as pltpu
from jax.experimental.pallas import tpu_sc as plsc

_N = 50000
_E = 800000
_HID = 128
_DEPTH = 9
_NB = 4
_FB = 32
_NS = 16
_NC = 2
_CHUNK = 96
_FIRE = 1
_WAVE = _CHUNK * _FIRE
_EPAD = 811008
_NWAVES = _EPAD // _WAVE
_W4 = 528
_W1 = 264
_NPAD = 50048
_RPT = _NPAD // _NS
_EPS = 1e-5
_R = 1000
_GRID = _N // _R

_f32 = jnp.float32
_i32 = jnp.int32

_MESH = plsc.VectorSubcoreMesh(core_axis_name="c", subcore_axis_name="s")


def _edge_waves(rows_h, cols_h, base, bufs, acc, gtab, npairs):
    (rbA, cbA, gbA, gsA, ssA), (rbB, cbB, gbB, gsB, ssB) = bufs

    def idx_load(w, rb, cb):
        pltpu.sync_copy(rows_h.at[w], rb)
        pltpu.sync_copy(cols_h.at[w], cb)

    def g_fire(rb, gb, sem):
        for j in range(_FIRE):
            pltpu.async_copy(gtab.at[rb.at[j]], gb.at[j], sem)

    def g_wait(rb, gb, sem):
        for j in range(_FIRE):
            pltpu.make_async_copy(gtab.at[rb.at[j]], gb.at[j], sem).wait()

    def s_fire(cb, gb, sem):
        pass

    def s_wait(cb, gb, sem):
        pass

    idx_load(base, rbA, cbA)
    g_fire(rbA, gbA, gsA)

    def body(k, carry):
        wA = base + 2 * k
        g_wait(rbA, gbA, gsA)
        s_fire(cbA, gbA, ssA)

        @pl.when(k > 0)
        def _():
            s_wait(cbB, gbB, ssB)
        idx_load(wA + 1, rbB, cbB)
        g_fire(rbB, gbB, gsB)
        g_wait(rbB, gbB, gsB)
        s_fire(cbB, gbB, ssB)

        @pl.when(k < npairs - 1)
        def _():
            s_wait(cbA, gbA, ssA)
            idx_load(wA + 2, rbA, cbA)
            g_fire(rbA, gbA, gsA)
        return carry

    lax.fori_loop(0, npairs, body, 0)
    s_wait(cbA, gbA, ssA)
    s_wait(cbB, gbB, ssB)


def _sc_conv4_body(g0, g1, g2, g3, rows_h, cols_h, wide,
                   s0, s1, s2, s3,
                   rbA, cbA, gbA, rbB, cbB, gbB, acc, gsA, ssA, gsB, ssB):
    cid = lax.axis_index("c")
    sid = lax.axis_index("s")
    bufs = ((rbA, cbA, gbA, gsA, ssA), (rbB, cbB, gbB, gsB, ssB))
    g_refs = (g0, g1, g2, g3)
    s_refs = (s0, s1, s2, s3)
    for b in range(_NB):
        @pl.when(cid == (b % _NC))
        def _(b=b):
            g = g_refs[b]
            s = s_refs[b]
            pltpu.sync_copy(g.at[pl.ds(sid * _RPT, _RPT)],
                            acc.at[pl.ds(sid * _RPT, _RPT)])
            plsc.subcore_barrier()
            _edge_waves(rows_h, cols_h, sid * _W4, bufs, acc, wide, _W4 // 2)
            plsc.subcore_barrier()
            pltpu.sync_copy(acc.at[pl.ds(sid * _RPT, _RPT)],
                            s.at[pl.ds(sid * _RPT, _RPT)])
            plsc.subcore_barrier()


def _sc_conv1_body(g0, rows_h, cols_h, wide, sp,
                   rbA, cbA, gbA, rbB, cbB, gbB, acc, gsA, ssA, gsB, ssB):
    cid = lax.axis_index("c")
    sid = lax.axis_index("s")
    bufs = ((rbA, cbA, gbA, gsA, ssA), (rbB, cbB, gbB, gsB, ssB))
    pltpu.sync_copy(g0.at[pl.ds(sid * _RPT, _RPT)],
                    acc.at[pl.ds(sid * _RPT, _RPT)])
    plsc.subcore_barrier()
    _edge_waves(rows_h, cols_h, (sid * _NC + cid) * _W1, bufs, acc, wide,
                _W1 // 2)
    plsc.subcore_barrier()
    pltpu.sync_copy(acc.at[pl.ds(sid * _RPT, _RPT)],
                    sp.at[cid, pl.ds(sid * _RPT, _RPT)])


_SC_SCRATCH = [
    pltpu.VMEM((_FIRE, _CHUNK), _i32),
    pltpu.VMEM((_FIRE, _CHUNK), _i32),
    pltpu.VMEM((_FIRE, _CHUNK, _HID), _f32),
    pltpu.VMEM((_FIRE, _CHUNK), _i32),
    pltpu.VMEM((_FIRE, _CHUNK), _i32),
    pltpu.VMEM((_FIRE, _CHUNK, _HID), _f32),
    pltpu.VMEM_SHARED((_NPAD, _FB), _f32),
    pltpu.SemaphoreType.DMA,
    pltpu.SemaphoreType.DMA,
    pltpu.SemaphoreType.DMA,
    pltpu.SemaphoreType.DMA,
]

_SC_PARAMS = pltpu.CompilerParams(use_tc_tiling_on_sc=False)

_sc_conv4 = pl.kernel(
    _sc_conv4_body,
    out_type=[jax.ShapeDtypeStruct((_NPAD, _FB), _f32)] * _NB,
    mesh=_MESH,
    scratch_types=_SC_SCRATCH,
    compiler_params=_SC_PARAMS,
)

_sc_conv1 = pl.kernel(
    _sc_conv1_body,
    out_type=jax.ShapeDtypeStruct((_NC, _NPAD, _FB), _f32),
    mesh=_MESH,
    scratch_types=_SC_SCRATCH,
    compiler_params=_SC_PARAMS,
)


def _dis_body(spdeg_ref, dis_ref):
    deg = spdeg_ref[0, :, 0:1] + spdeg_ref[1, :, 0:1] - 1.0
    dis_ref[...] = jnp.broadcast_to(lax.rsqrt(deg), (_R, _HID))


def _dis_call(spdeg):
    return pl.pallas_call(
        _dis_body,
        grid=(_GRID,),
        in_specs=[pl.BlockSpec((_NC, _R, _FB), lambda i: (0, i, 0))],
        out_specs=pl.BlockSpec((_R, _HID), lambda i: (i, 0)),
        out_shape=jax.ShapeDtypeStruct((_N, _HID), _f32),
    )(spdeg)


def _first_body(x_ref, dis_ref, w_ref, *g_refs):
    m = x_ref[...] * dis_ref[:, 0:x_ref.shape[1]]
    g = jnp.dot(m, w_ref[...], preferred_element_type=_f32)
    for b in range(_NB):
        g_refs[b][...] = g[:, b * _FB:(b + 1) * _FB]


def _first_call(x, dis, w):
    return pl.pallas_call(
        _first_body,
        grid=(_GRID,),
        in_specs=[
            pl.BlockSpec((_R, x.shape[1]), lambda i: (i, 0)),
            pl.BlockSpec((_R, _HID), lambda i: (i, 0)),
            pl.BlockSpec(w.shape, lambda i: (0, 0)),
        ],
        out_specs=[pl.BlockSpec((_R, _FB), lambda i: (i, 0))] * _NB,
        out_shape=[jax.ShapeDtypeStruct((_NPAD, _FB), _f32)] * _NB,
    )(x, dis, w)


def _stats_body(s0, s1, s2, s3, dis_ref, out_ref):
    @pl.when(pl.program_id(0) == 0)
    def _():
        out_ref[...] = jnp.zeros_like(out_ref)

    dis = dis_ref[:, 0:_FB]
    su, ss = [], []
    for s in (s0, s1, s2, s3):
        c = s[...] * dis
        su.append(jnp.sum(c, axis=0))
        ss.append(jnp.sum(c * c, axis=0))
    out_ref[0, 0, :] += jnp.concatenate(su, axis=0)
    out_ref[1, 0, :] += jnp.concatenate(ss, axis=0)


def _stats_call(s_blocks, dis):
    return pl.pallas_call(
        _stats_body,
        grid=(_GRID,),
        in_specs=[pl.BlockSpec((_R, _FB), lambda i: (i, 0))] * _NB
        + [pl.BlockSpec((_R, _HID), lambda i: (i, 0))],
        out_specs=pl.BlockSpec((2, 1, _HID), lambda i: (0, 0, 0)),
        out_shape=jax.ShapeDtypeStruct((2, 1, _HID), _f32),
    )(*s_blocks, dis)


def _fuse_body(save_h, use_res, nb_out, *refs):
    refs = list(refs)
    s0, s1, s2, s3, dis_ref, sums_ref, gam_ref, bet_ref, w_ref = refs[:9]
    refs = refs[9:]
    res_ref = refs.pop(0) if use_res else None
    g_refs = refs[:nb_out]
    h_ref = refs[nb_out] if save_h else None

    s = jnp.concatenate([s0[...], s1[...], s2[...], s3[...]], axis=1)
    dis = dis_ref[...]
    c = s * dis
    mu = sums_ref[0, 0, :] * (1.0 / _N)
    var = sums_ref[1, 0, :] * (1.0 / _N) - mu * mu
    inv = lax.rsqrt(var + _EPS)
    a = gam_ref[0, :] * inv
    b0 = bet_ref[0, :] - mu * a
    h = jnp.maximum(c * a + b0, 0.0)
    if save_h:
        h_ref[...] = h
    if use_res:
        h = h + res_ref[...]
    m = h * dis
    g = jnp.dot(m, w_ref[...], preferred_element_type=_f32)
    for b in range(nb_out):
        g_refs[b][...] = g[:, b * _FB:(b + 1) * _FB]


def _fuse_call(s_blocks, dis, sums, gamma, beta, w, res=None, save_h=False):
    use_res = res is not None
    nb_out = w.shape[1] // _FB
    ins = list(s_blocks) + [dis, sums, gamma, beta, w]
    in_specs = (
        [pl.BlockSpec((_R, _FB), lambda i: (i, 0))] * _NB
        + [
            pl.BlockSpec((_R, _HID), lambda i: (i, 0)),
            pl.BlockSpec((2, 1, _HID), lambda i: (0, 0, 0)),
            pl.BlockSpec((1, _HID), lambda i: (0, 0)),
            pl.BlockSpec((1, _HID), lambda i: (0, 0)),
            pl.BlockSpec(w.shape, lambda i: (0, 0)),
        ]
    )
    if use_res:
        ins.append(res)
        in_specs.append(pl.BlockSpec((_R, _HID), lambda i: (i, 0)))
    out_specs = [pl.BlockSpec((_R, _FB), lambda i: (i, 0))] * nb_out
    out_shape = [jax.ShapeDtypeStruct((_NPAD, _FB), _f32)] * nb_out
    if save_h:
        out_specs.append(pl.BlockSpec((_R, _HID), lambda i: (i, 0)))
        out_shape.append(jax.ShapeDtypeStruct((_N, _HID), _f32))
    body = functools.partial(_fuse_body, save_h, use_res, nb_out)
    return pl.pallas_call(
        body,
        grid=(_GRID,),
        in_specs=in_specs,
        out_specs=out_specs,
        out_shape=out_shape,
    )(*ins)


def _final_body(sp_ref, g_ref, dis_ref, out_ref):
    v = (sp_ref[0, :, 0:1] + sp_ref[1, :, 0:1] - g_ref[:, 0:1]) * dis_ref[:, 0:1]
    out_ref[...] = jax.nn.sigmoid(v)


def _final_call(sp, g0, dis):
    return pl.pallas_call(
        _final_body,
        grid=(_GRID,),
        in_specs=[
            pl.BlockSpec((_NC, _R, _FB), lambda i: (0, i, 0)),
            pl.BlockSpec((_R, _FB), lambda i: (i, 0)),
            pl.BlockSpec((_R, _HID), lambda i: (i, 0)),
        ],
        out_specs=pl.BlockSpec((_R, 1), lambda i: (i, 0)),
        out_shape=jax.ShapeDtypeStruct((_N, 1), _f32),
    )(sp, g0, dis)


def kernel(x, edge_index, W_in, W_h1, W_h2, W_out,
           bn1_gamma, bn1_beta, bn2_gamma, bn2_beta):
    npadE = _EPAD - _E
    rows = jnp.concatenate(
        [edge_index[0], (jnp.arange(npadE, dtype=_i32) * 37) % _N])
    cols = jnp.concatenate(
        [edge_index[1], _N + (jnp.arange(npadE, dtype=_i32) % 16)])
    rows = rows.reshape(_NWAVES, _FIRE, _CHUNK)
    cols = cols.reshape(_NWAVES, _FIRE, _CHUNK)

    ones_tab = jnp.ones((_NPAD, _FB), _f32)
    spdeg = _sc_conv1(ones_tab, rows, cols, jnp.ones((_N, _HID), _f32))
    dis = _dis_call(spdeg)

    bn1_gamma = bn1_gamma.reshape(_DEPTH + 1, 1, _HID)
    bn1_beta = bn1_beta.reshape(_DEPTH + 1, 1, _HID)
    bn2_gamma = bn2_gamma.reshape(_DEPTH - 1, 1, _HID)
    bn2_beta = bn2_beta.reshape(_DEPTH - 1, 1, _HID)
    w_out_pad = jnp.pad(W_out, ((0, 0), (0, _FB - W_out.shape[1])))

    g = _first_call(x, dis, W_in)
    xs = []
    for k in range(18):
        s = _sc_conv4(*g, rows, cols, dis)
        sums = _stats_call(s, dis)
        if k <= 9:
            gamma, beta = bn1_gamma[k], bn1_beta[k]
        else:
            gamma, beta = bn2_gamma[k - 10], bn2_beta[k - 10]
        w_next = (W_h1[k] if k <= 8 else
                  (W_h2[k - 9] if k <= 16 else w_out_pad))
        res = xs[17 - k] if k >= 9 else None
        save_h = k <= 8
        outs = _fuse_call(s, dis, sums, gamma, beta, w_next,
                          res=res, save_h=save_h)
        if save_h:
            g, h = outs[:-1], outs[-1]
            xs.append(h)
        else:
            g = outs
        if k == 17:
            g_last = g[0]
    sp = _sc_conv1(g_last, rows, cols, dis)
    return _final_call(sp, g_last, dis)


# speedup vs baseline: 1.9127x; 1.9127x over previous
import functools

import jax
import jax.numpy as jnp
from jax import lax
from jax.experimental import pallas as pl
from jax.experimental.pallas import tpu as pltpu
from jax.experimental.pallas import tpu_sc as plsc

_N = 50000
_E = 800000
_HID = 128
_DEPTH = 9
_NB = 8
_FB = 16
_NS = 16
_NC = 2
_CHUNK = 384
_EPAD = 811008
_NWAVES = _EPAD // _CHUNK
_W4 = 132
_W1 = 66
_NPAD = 50048
_RPT = _NPAD // _NS
_EPS = 1e-5
_R = 1000
_GRID = _N // _R

_f32 = jnp.float32
_i32 = jnp.int32

_MESH = plsc.VectorSubcoreMesh(core_axis_name="c", subcore_axis_name="s")


def _edge_waves(rows_h, cols_h, base, bufs, acc, gtab, npairs):
    (rbA, cbA, gbA, gsA, ssA), (rbB, cbB, gbB, gsB, ssB) = bufs

    def idx_load(w, rb, cb):
        pltpu.sync_copy(rows_h.at[w], rb)
        pltpu.sync_copy(cols_h.at[w], cb)

    def g_fire(rb, gb, sem):
        pltpu.async_copy(gtab.at[rb], gb, sem)

    def g_wait(rb, gb, sem):
        pltpu.make_async_copy(gtab.at[rb], gb, sem).wait()

    def s_fire(cb, gb, sem):
        pltpu.async_copy(gb, acc.at[cb], sem, add=True)

    def s_wait(cb, gb, sem):
        pltpu.make_async_copy(gb, acc.at[cb], sem).wait()

    idx_load(base, rbA, cbA)
    g_fire(rbA, gbA, gsA)

    def body(k, carry):
        wA = base + 2 * k
        g_wait(rbA, gbA, gsA)
        s_fire(cbA, gbA, ssA)

        @pl.when(k > 0)
        def _():
            s_wait(cbB, gbB, ssB)
        idx_load(wA + 1, rbB, cbB)
        g_fire(rbB, gbB, gsB)
        g_wait(rbB, gbB, gsB)
        s_fire(cbB, gbB, ssB)

        @pl.when(k < npairs - 1)
        def _():
            s_wait(cbA, gbA, ssA)
            idx_load(wA + 2, rbA, cbA)
            g_fire(rbA, gbA, gsA)
        return carry

    lax.fori_loop(0, npairs, body, 0)
    s_wait(cbA, gbA, ssA)
    s_wait(cbB, gbB, ssB)


def _stage_block(g, gtab, acc, sid):
    sl = pl.ds(sid * _RPT, _RPT)
    pltpu.sync_copy(g.at[sl], gtab.at[sl])
    pltpu.sync_copy(g.at[sl], acc.at[sl])
    plsc.subcore_barrier()


def _sc_conv8_body(g0, g1, g2, g3, g4, g5, g6, g7, rows_h, cols_h,
                   s0, s1, s2, s3, s4, s5, s6, s7,
                   rbA, cbA, gbA, rbB, cbB, gbB, gtab, acc,
                   gsA, ssA, gsB, ssB):
    cid = lax.axis_index("c")
    sid = lax.axis_index("s")
    bufs = ((rbA, cbA, gbA, gsA, ssA), (rbB, cbB, gbB, gsB, ssB))
    g_refs = (g0, g1, g2, g3, g4, g5, g6, g7)
    s_refs = (s0, s1, s2, s3, s4, s5, s6, s7)
    for b in range(_NB):
        @pl.when(cid == (b % _NC))
        def _(b=b):
            _stage_block(g_refs[b], gtab, acc, sid)
            _edge_waves(rows_h, cols_h, sid * _W4, bufs, acc, gtab, _W4 // 2)
            plsc.subcore_barrier()
            pltpu.sync_copy(acc.at[pl.ds(sid * _RPT, _RPT)],
                            s_refs[b].at[pl.ds(sid * _RPT, _RPT)])
            plsc.subcore_barrier()


def _sc_conv1_body(g0, rows_h, cols_h, sp,
                   rbA, cbA, gbA, rbB, cbB, gbB, gtab, acc,
                   gsA, ssA, gsB, ssB):
    cid = lax.axis_index("c")
    sid = lax.axis_index("s")
    bufs = ((rbA, cbA, gbA, gsA, ssA), (rbB, cbB, gbB, gsB, ssB))
    _stage_block(g0, gtab, acc, sid)
    _edge_waves(rows_h, cols_h, (sid * _NC + cid) * _W1, bufs, acc, gtab,
                _W1 // 2)
    plsc.subcore_barrier()
    pltpu.sync_copy(acc.at[pl.ds(sid * _RPT, _RPT)],
                    sp.at[cid, pl.ds(sid * _RPT, _RPT)])


_SC_SCRATCH = [
    pltpu.VMEM((_CHUNK,), _i32),
    pltpu.VMEM((_CHUNK,), _i32),
    pltpu.VMEM((_CHUNK, _FB), _f32),
    pltpu.VMEM((_CHUNK,), _i32),
    pltpu.VMEM((_CHUNK,), _i32),
    pltpu.VMEM((_CHUNK, _FB), _f32),
    pltpu.VMEM_SHARED((_NPAD, _FB), _f32),
    pltpu.VMEM_SHARED((_NPAD, _FB), _f32),
    pltpu.SemaphoreType.DMA,
    pltpu.SemaphoreType.DMA,
    pltpu.SemaphoreType.DMA,
    pltpu.SemaphoreType.DMA,
]

_SC_PARAMS = pltpu.CompilerParams(use_tc_tiling_on_sc=False)

_sc_conv8 = pl.kernel(
    _sc_conv8_body,
    out_type=[jax.ShapeDtypeStruct((_NPAD, _FB), _f32)] * _NB,
    mesh=_MESH,
    scratch_types=_SC_SCRATCH,
    compiler_params=_SC_PARAMS,
)

_sc_conv1 = pl.kernel(
    _sc_conv1_body,
    out_type=jax.ShapeDtypeStruct((_NC, _NPAD, _FB), _f32),
    mesh=_MESH,
    scratch_types=_SC_SCRATCH,
    compiler_params=_SC_PARAMS,
)


def _dis_body(spdeg_ref, dis_ref):
    deg = spdeg_ref[0, :, 0:1] + spdeg_ref[1, :, 0:1] - 1.0
    dis_ref[...] = jnp.broadcast_to(lax.rsqrt(deg), (_R, _HID))


def _dis_call(spdeg):
    return pl.pallas_call(
        _dis_body,
        grid=(_GRID,),
        in_specs=[pl.BlockSpec((_NC, _R, _FB), lambda i: (0, i, 0))],
        out_specs=pl.BlockSpec((_R, _HID), lambda i: (i, 0)),
        out_shape=jax.ShapeDtypeStruct((_N, _HID), _f32),
    )(spdeg)


def _first_body(x_ref, dis_ref, w_ref, *g_refs):
    m = x_ref[...] * dis_ref[:, 0:x_ref.shape[1]]
    g = jnp.dot(m, w_ref[...], preferred_element_type=_f32)
    for b in range(_NB):
        g_refs[b][...] = g[:, b * _FB:(b + 1) * _FB]


def _first_call(x, dis, w):
    return pl.pallas_call(
        _first_body,
        grid=(_GRID,),
        in_specs=[
            pl.BlockSpec((_R, x.shape[1]), lambda i: (i, 0)),
            pl.BlockSpec((_R, _HID), lambda i: (i, 0)),
            pl.BlockSpec(w.shape, lambda i: (0, 0)),
        ],
        out_specs=[pl.BlockSpec((_R, _FB), lambda i: (i, 0))] * _NB,
        out_shape=[jax.ShapeDtypeStruct((_NPAD, _FB), _f32)] * _NB,
    )(x, dis, w)


def _stats_body(*refs):
    s_refs = refs[:_NB]
    dis_ref, out_ref = refs[_NB], refs[_NB + 1]

    @pl.when(pl.program_id(0) == 0)
    def _():
        out_ref[...] = jnp.zeros_like(out_ref)

    dis = dis_ref[:, 0:_FB]
    su, ss = [], []
    for s in s_refs:
        c = s[...] * dis
        su.append(jnp.sum(c, axis=0))
        ss.append(jnp.sum(c * c, axis=0))
    out_ref[0, 0, :] += jnp.concatenate(su, axis=0)
    out_ref[1, 0, :] += jnp.concatenate(ss, axis=0)


def _stats_call(s_blocks, dis):
    return pl.pallas_call(
        _stats_body,
        grid=(_GRID,),
        in_specs=[pl.BlockSpec((_R, _FB), lambda i: (i, 0))] * _NB
        + [pl.BlockSpec((_R, _HID), lambda i: (i, 0))],
        out_specs=pl.BlockSpec((2, 1, _HID), lambda i: (0, 0, 0)),
        out_shape=jax.ShapeDtypeStruct((2, 1, _HID), _f32),
    )(*s_blocks, dis)


def _fuse_body(save_h, use_res, nb_out, *refs):
    refs = list(refs)
    s_refs = refs[:_NB]
    dis_ref, sums_ref, gam_ref, bet_ref, w_ref = refs[_NB:_NB + 5]
    refs = refs[_NB + 5:]
    res_ref = refs.pop(0) if use_res else None
    g_refs = refs[:nb_out]
    h_ref = refs[nb_out] if save_h else None

    s = jnp.concatenate([r[...] for r in s_refs], axis=1)
    dis = dis_ref[...]
    c = s * dis
    mu = sums_ref[0, 0, :] * (1.0 / _N)
    var = sums_ref[1, 0, :] * (1.0 / _N) - mu * mu
    inv = lax.rsqrt(var + _EPS)
    a = gam_ref[0, :] * inv
    b0 = bet_ref[0, :] - mu * a
    h = jnp.maximum(c * a + b0, 0.0)
    if save_h:
        h_ref[...] = h
    if use_res:
        h = h + res_ref[...]
    m = h * dis
    g = jnp.dot(m, w_ref[...], preferred_element_type=_f32)
    for b in range(nb_out):
        g_refs[b][...] = g[:, b * _FB:(b + 1) * _FB]


def _fuse_call(s_blocks, dis, sums, gamma, beta, w, res=None, save_h=False):
    use_res = res is not None
    nb_out = w.shape[1] // _FB
    ins = list(s_blocks) + [dis, sums, gamma, beta, w]
    in_specs = (
        [pl.BlockSpec((_R, _FB), lambda i: (i, 0))] * _NB
        + [
            pl.BlockSpec((_R, _HID), lambda i: (i, 0)),
            pl.BlockSpec((2, 1, _HID), lambda i: (0, 0, 0)),
            pl.BlockSpec((1, _HID), lambda i: (0, 0)),
            pl.BlockSpec((1, _HID), lambda i: (0, 0)),
            pl.BlockSpec(w.shape, lambda i: (0, 0)),
        ]
    )
    if use_res:
        ins.append(res)
        in_specs.append(pl.BlockSpec((_R, _HID), lambda i: (i, 0)))
    out_specs = [pl.BlockSpec((_R, _FB), lambda i: (i, 0))] * nb_out
    out_shape = [jax.ShapeDtypeStruct((_NPAD, _FB), _f32)] * nb_out
    if save_h:
        out_specs.append(pl.BlockSpec((_R, _HID), lambda i: (i, 0)))
        out_shape.append(jax.ShapeDtypeStruct((_N, _HID), _f32))
    body = functools.partial(_fuse_body, save_h, use_res, nb_out)
    return pl.pallas_call(
        body,
        grid=(_GRID,),
        in_specs=in_specs,
        out_specs=out_specs,
        out_shape=out_shape,
    )(*ins)


def _final_body(sp_ref, g_ref, dis_ref, out_ref):
    v = (sp_ref[0, :, 0:1] + sp_ref[1, :, 0:1] - g_ref[:, 0:1]) * dis_ref[:, 0:1]
    out_ref[...] = jax.nn.sigmoid(v)


def _final_call(sp, g0, dis):
    return pl.pallas_call(
        _final_body,
        grid=(_GRID,),
        in_specs=[
            pl.BlockSpec((_NC, _R, _FB), lambda i: (0, i, 0)),
            pl.BlockSpec((_R, _FB), lambda i: (i, 0)),
            pl.BlockSpec((_R, _HID), lambda i: (i, 0)),
        ],
        out_specs=pl.BlockSpec((_R, 1), lambda i: (i, 0)),
        out_shape=jax.ShapeDtypeStruct((_N, 1), _f32),
    )(sp, g0, dis)


def kernel(x, edge_index, W_in, W_h1, W_h2, W_out,
           bn1_gamma, bn1_beta, bn2_gamma, bn2_beta):
    npadE = _EPAD - _E
    rows = jnp.concatenate(
        [edge_index[0], (jnp.arange(npadE, dtype=_i32) * 37) % _N])
    cols = jnp.concatenate(
        [edge_index[1], _N + (jnp.arange(npadE, dtype=_i32) % 16)])
    rows = rows.reshape(_NWAVES, _CHUNK)
    cols = cols.reshape(_NWAVES, _CHUNK)

    ones_tab = jnp.ones((_NPAD, _FB), _f32)
    spdeg = _sc_conv1(ones_tab, rows, cols)
    dis = _dis_call(spdeg)

    bn1_gamma = bn1_gamma.reshape(_DEPTH + 1, 1, _HID)
    bn1_beta = bn1_beta.reshape(_DEPTH + 1, 1, _HID)
    bn2_gamma = bn2_gamma.reshape(_DEPTH - 1, 1, _HID)
    bn2_beta = bn2_beta.reshape(_DEPTH - 1, 1, _HID)
    w_out_pad = jnp.pad(W_out, ((0, 0), (0, _FB - W_out.shape[1])))

    g = _first_call(x, dis, W_in)
    xs = []
    g_last = None
    for k in range(18):
        s = _sc_conv8(*g, rows, cols)
        sums = _stats_call(s, dis)
        if k <= 9:
            gamma, beta = bn1_gamma[k], bn1_beta[k]
        else:
            gamma, beta = bn2_gamma[k - 10], bn2_beta[k - 10]
        w_next = (W_h1[k] if k <= 8 else
                  (W_h2[k - 9] if k <= 16 else w_out_pad))
        res = xs[17 - k] if k >= 9 else None
        save_h = k <= 8
        outs = _fuse_call(s, dis, sums, gamma, beta, w_next,
                          res=res, save_h=save_h)
        if save_h:
            g, h = outs[:-1], outs[-1]
            xs.append(h)
        else:
            g = outs
        if k == 17:
            g_last = g[0]
    sp = _sc_conv1(g_last, rows, cols)
    return _final_call(sp, g_last, dis)


# speedup vs baseline: 2.7124x; 1.4181x over previous
import functools

import jax
import jax.numpy as jnp
from jax import lax
from jax.experimental import pallas as pl
from jax.experimental.pallas import tpu as pltpu
from jax.experimental.pallas import tpu_sc as plsc

_N = 50000
_E = 800000
_HID = 128
_DEPTH = 9
_NB = 4
_FB = 32
_NS = 16
_NC = 2
_CHUNK = 384
_FIRE = 1
_WAVE = _CHUNK * _FIRE
_EPAD = 811008
_NWAVES = _EPAD // _WAVE
_W4 = 132
_W1 = 66
_NPAD = 50048
_RPT = _NPAD // _NS
_EPS = 1e-5
_R = 1000
_GRID = _N // _R

_f32 = jnp.float32
_i32 = jnp.int32

_MESH = plsc.VectorSubcoreMesh(core_axis_name="c", subcore_axis_name="s")


def _edge_waves(rows_h, cols_h, base, bufs, acc, gtab, npairs):
    (rbA, cbA, gbA, gsA, ssA), (rbB, cbB, gbB, gsB, ssB) = bufs

    def idx_load(w, rb, cb):
        pltpu.sync_copy(rows_h.at[w], rb)
        pltpu.sync_copy(cols_h.at[w], cb)

    def g_fire(rb, gb, sem):
        for j in range(_FIRE):
            pltpu.async_copy(gtab.at[rb.at[j]], gb.at[j], sem)

    def g_wait(rb, gb, sem):
        for j in range(_FIRE):
            pltpu.make_async_copy(gtab.at[rb.at[j]], gb.at[j], sem).wait()

    def s_fire(cb, gb, sem):
        for j in range(_FIRE):
            pltpu.async_copy(gb.at[j], acc.at[cb.at[j]], sem, add=True)

    def s_wait(cb, gb, sem):
        for j in range(_FIRE):
            pltpu.make_async_copy(gb.at[j], acc.at[cb.at[j]], sem).wait()

    idx_load(base, rbA, cbA)
    g_fire(rbA, gbA, gsA)

    def body(k, carry):
        wA = base + 2 * k
        g_wait(rbA, gbA, gsA)
        s_fire(cbA, gbA, ssA)

        @pl.when(k > 0)
        def _():
            s_wait(cbB, gbB, ssB)
        idx_load(wA + 1, rbB, cbB)
        g_fire(rbB, gbB, gsB)
        g_wait(rbB, gbB, gsB)
        s_fire(cbB, gbB, ssB)

        @pl.when(k < npairs - 1)
        def _():
            s_wait(cbA, gbA, ssA)
            idx_load(wA + 2, rbA, cbA)
            g_fire(rbA, gbA, gsA)
        return carry

    lax.fori_loop(0, npairs, body, 0)
    s_wait(cbA, gbA, ssA)
    s_wait(cbB, gbB, ssB)


def _sc_conv4_body(g0, g1, g2, g3, rows_h, cols_h,
                   s0, s1, s2, s3,
                   rbA, cbA, gbA, rbB, cbB, gbB, acc, gsA, ssA, gsB, ssB):
    cid = lax.axis_index("c")
    sid = lax.axis_index("s")
    bufs = ((rbA, cbA, gbA, gsA, ssA), (rbB, cbB, gbB, gsB, ssB))
    g_refs = (g0, g1, g2, g3)
    s_refs = (s0, s1, s2, s3)
    for b in range(_NB):
        @pl.when(cid == (b % _NC))
        def _(b=b):
            g = g_refs[b]
            s = s_refs[b]
            pltpu.sync_copy(g.at[pl.ds(sid * _RPT, _RPT)],
                            acc.at[pl.ds(sid * _RPT, _RPT)])
            plsc.subcore_barrier()
            _edge_waves(rows_h, cols_h, sid * _W4, bufs, acc, g, _W4 // 2)
            plsc.subcore_barrier()
            pltpu.sync_copy(acc.at[pl.ds(sid * _RPT, _RPT)],
                            s.at[pl.ds(sid * _RPT, _RPT)])
            plsc.subcore_barrier()


def _sc_conv1_body(g0, rows_h, cols_h, sp,
                   rbA, cbA, gbA, rbB, cbB, gbB, acc, gsA, ssA, gsB, ssB):
    cid = lax.axis_index("c")
    sid = lax.axis_index("s")
    bufs = ((rbA, cbA, gbA, gsA, ssA), (rbB, cbB, gbB, gsB, ssB))
    pltpu.sync_copy(g0.at[pl.ds(sid * _RPT, _RPT)],
                    acc.at[pl.ds(sid * _RPT, _RPT)])
    plsc.subcore_barrier()
    _edge_waves(rows_h, cols_h, (sid * _NC + cid) * _W1, bufs, acc, g0,
                _W1 // 2)
    plsc.subcore_barrier()
    pltpu.sync_copy(acc.at[pl.ds(sid * _RPT, _RPT)],
                    sp.at[cid, pl.ds(sid * _RPT, _RPT)])


_SC_SCRATCH = [
    pltpu.VMEM((_FIRE, _CHUNK), _i32),
    pltpu.VMEM((_FIRE, _CHUNK), _i32),
    pltpu.VMEM((_FIRE, _CHUNK, _FB), _f32),
    pltpu.VMEM((_FIRE, _CHUNK), _i32),
    pltpu.VMEM((_FIRE, _CHUNK), _i32),
    pltpu.VMEM((_FIRE, _CHUNK, _FB), _f32),
    pltpu.VMEM_SHARED((_NPAD, _FB), _f32),
    pltpu.SemaphoreType.DMA,
    pltpu.SemaphoreType.DMA,
    pltpu.SemaphoreType.DMA,
    pltpu.SemaphoreType.DMA,
]

_SC_PARAMS = pltpu.CompilerParams(use_tc_tiling_on_sc=False)

_sc_conv4 = pl.kernel(
    _sc_conv4_body,
    out_type=[jax.ShapeDtypeStruct((_NPAD, _FB), _f32)] * _NB,
    mesh=_MESH,
    scratch_types=_SC_SCRATCH,
    compiler_params=_SC_PARAMS,
)

_sc_conv1 = pl.kernel(
    _sc_conv1_body,
    out_type=jax.ShapeDtypeStruct((_NC, _NPAD, _FB), _f32),
    mesh=_MESH,
    scratch_types=_SC_SCRATCH,
    compiler_params=_SC_PARAMS,
)


def _dis_body(spdeg_ref, dis_ref):
    deg = spdeg_ref[0, :, 0:1] + spdeg_ref[1, :, 0:1] - 1.0
    dis_ref[...] = jnp.broadcast_to(lax.rsqrt(deg), (_R, _HID))


def _dis_call(spdeg):
    return pl.pallas_call(
        _dis_body,
        grid=(_GRID,),
        in_specs=[pl.BlockSpec((_NC, _R, _FB), lambda i: (0, i, 0))],
        out_specs=pl.BlockSpec((_R, _HID), lambda i: (i, 0)),
        out_shape=jax.ShapeDtypeStruct((_N, _HID), _f32),
    )(spdeg)


def _first_body(x_ref, dis_ref, w_ref, *g_refs):
    m = x_ref[...] * dis_ref[:, 0:x_ref.shape[1]]
    g = jnp.dot(m, w_ref[...], preferred_element_type=_f32)
    for b in range(_NB):
        g_refs[b][...] = g[:, b * _FB:(b + 1) * _FB]


def _first_call(x, dis, w):
    return pl.pallas_call(
        _first_body,
        grid=(_GRID,),
        in_specs=[
            pl.BlockSpec((_R, x.shape[1]), lambda i: (i, 0)),
            pl.BlockSpec((_R, _HID), lambda i: (i, 0)),
            pl.BlockSpec(w.shape, lambda i: (0, 0)),
        ],
        out_specs=[pl.BlockSpec((_R, _FB), lambda i: (i, 0))] * _NB,
        out_shape=[jax.ShapeDtypeStruct((_NPAD, _FB), _f32)] * _NB,
    )(x, dis, w)


def _stats_body(s0, s1, s2, s3, dis_ref, out_ref):
    @pl.when(pl.program_id(0) == 0)
    def _():
        out_ref[...] = jnp.zeros_like(out_ref)

    dis = dis_ref[:, 0:_FB]
    su, ss = [], []
    for s in (s0, s1, s2, s3):
        c = s[...] * dis
        su.append(jnp.sum(c, axis=0))
        ss.append(jnp.sum(c * c, axis=0))
    out_ref[0, 0, :] += jnp.concatenate(su, axis=0)
    out_ref[1, 0, :] += jnp.concatenate(ss, axis=0)


def _stats_call(s_blocks, dis):
    return pl.pallas_call(
        _stats_body,
        grid=(_GRID,),
        in_specs=[pl.BlockSpec((_R, _FB), lambda i: (i, 0))] * _NB
        + [pl.BlockSpec((_R, _HID), lambda i: (i, 0))],
        out_specs=pl.BlockSpec((2, 1, _HID), lambda i: (0, 0, 0)),
        out_shape=jax.ShapeDtypeStruct((2, 1, _HID), _f32),
    )(*s_blocks, dis)


def _fuse_body(save_h, use_res, nb_out, *refs):
    refs = list(refs)
    s0, s1, s2, s3, dis_ref, sums_ref, gam_ref, bet_ref, w_ref = refs[:9]
    refs = refs[9:]
    res_ref = refs.pop(0) if use_res else None
    g_refs = refs[:nb_out]
    h_ref = refs[nb_out] if save_h else None

    s = jnp.concatenate([s0[...], s1[...], s2[...], s3[...]], axis=1)
    dis = dis_ref[...]
    c = s * dis
    mu = sums_ref[0, 0, :] * (1.0 / _N)
    var = sums_ref[1, 0, :] * (1.0 / _N) - mu * mu
    inv = lax.rsqrt(var + _EPS)
    a = gam_ref[0, :] * inv
    b0 = bet_ref[0, :] - mu * a
    h = jnp.maximum(c * a + b0, 0.0)
    if save_h:
        h_ref[...] = h
    if use_res:
        h = h + res_ref[...]
    m = h * dis
    g = jnp.dot(m, w_ref[...], preferred_element_type=_f32)
    for b in range(nb_out):
        g_refs[b][...] = g[:, b * _FB:(b + 1) * _FB]


def _fuse_call(s_blocks, dis, sums, gamma, beta, w, res=None, save_h=False):
    use_res = res is not None
    nb_out = w.shape[1] // _FB
    ins = list(s_blocks) + [dis, sums, gamma, beta, w]
    in_specs = (
        [pl.BlockSpec((_R, _FB), lambda i: (i, 0))] * _NB
        + [
            pl.BlockSpec((_R, _HID), lambda i: (i, 0)),
            pl.BlockSpec((2, 1, _HID), lambda i: (0, 0, 0)),
            pl.BlockSpec((1, _HID), lambda i: (0, 0)),
            pl.BlockSpec((1, _HID), lambda i: (0, 0)),
            pl.BlockSpec(w.shape, lambda i: (0, 0)),
        ]
    )
    if use_res:
        ins.append(res)
        in_specs.append(pl.BlockSpec((_R, _HID), lambda i: (i, 0)))
    out_specs = [pl.BlockSpec((_R, _FB), lambda i: (i, 0))] * nb_out
    out_shape = [jax.ShapeDtypeStruct((_NPAD, _FB), _f32)] * nb_out
    if save_h:
        out_specs.append(pl.BlockSpec((_R, _HID), lambda i: (i, 0)))
        out_shape.append(jax.ShapeDtypeStruct((_N, _HID), _f32))
    body = functools.partial(_fuse_body, save_h, use_res, nb_out)
    return pl.pallas_call(
        body,
        grid=(_GRID,),
        in_specs=in_specs,
        out_specs=out_specs,
        out_shape=out_shape,
    )(*ins)


def _final_body(sp_ref, g_ref, dis_ref, out_ref):
    v = (sp_ref[0, :, 0:1] + sp_ref[1, :, 0:1] - g_ref[:, 0:1]) * dis_ref[:, 0:1]
    out_ref[...] = jax.nn.sigmoid(v)


def _final_call(sp, g0, dis):
    return pl.pallas_call(
        _final_body,
        grid=(_GRID,),
        in_specs=[
            pl.BlockSpec((_NC, _R, _FB), lambda i: (0, i, 0)),
            pl.BlockSpec((_R, _FB), lambda i: (i, 0)),
            pl.BlockSpec((_R, _HID), lambda i: (i, 0)),
        ],
        out_specs=pl.BlockSpec((_R, 1), lambda i: (i, 0)),
        out_shape=jax.ShapeDtypeStruct((_N, 1), _f32),
    )(sp, g0, dis)


def kernel(x, edge_index, W_in, W_h1, W_h2, W_out,
           bn1_gamma, bn1_beta, bn2_gamma, bn2_beta):
    npadE = _EPAD - _E
    rows = jnp.concatenate(
        [edge_index[0], (jnp.arange(npadE, dtype=_i32) * 37) % _N])
    cols = jnp.concatenate(
        [edge_index[1], _N + (jnp.arange(npadE, dtype=_i32) % 16)])
    rows = rows.reshape(_NWAVES, _FIRE, _CHUNK)
    cols = cols.reshape(_NWAVES, _FIRE, _CHUNK)

    ones_tab = jnp.ones((_NPAD, _FB), _f32)
    spdeg = _sc_conv1(ones_tab, rows, cols)
    dis = _dis_call(spdeg)

    bn1_gamma = bn1_gamma.reshape(_DEPTH + 1, 1, _HID)
    bn1_beta = bn1_beta.reshape(_DEPTH + 1, 1, _HID)
    bn2_gamma = bn2_gamma.reshape(_DEPTH - 1, 1, _HID)
    bn2_beta = bn2_beta.reshape(_DEPTH - 1, 1, _HID)
    w_out_pad = jnp.pad(W_out, ((0, 0), (0, _FB - W_out.shape[1])))

    g = _first_call(x, dis, W_in)
    xs = []
    for k in range(18):
        s = _sc_conv4(*g, rows, cols)
        sums = _stats_call(s, dis)
        if k <= 9:
            gamma, beta = bn1_gamma[k], bn1_beta[k]
        else:
            gamma, beta = bn2_gamma[k - 10], bn2_beta[k - 10]
        w_next = (W_h1[k] if k <= 8 else
                  (W_h2[k - 9] if k <= 16 else w_out_pad))
        res = xs[17 - k] if k >= 9 else None
        save_h = k <= 8
        outs = _fuse_call(s, dis, sums, gamma, beta, w_next,
                          res=res, save_h=save_h)
        if save_h:
            g, h = outs[:-1], outs[-1]
            xs.append(h)
        else:
            g = outs
        if k == 17:
            g_last = g[0]
    sp = _sc_conv1(g_last, rows, cols)
    return _final_call(sp, g_last, dis)


# speedup vs baseline: 3.0833x; 1.1367x over previous
import functools

import jax
import jax.numpy as jnp
from jax import lax
from jax.experimental import pallas as pl
from jax.experimental.pallas import tpu as pltpu
from jax.experimental.pallas import tpu_sc as plsc

_N = 50000
_E = 800000
_HID = 128
_DEPTH = 9
_NB = 4
_FB = 32
_NS = 16
_NC = 2
_CHUNK = 384
_FIRE = 1
_WAVE = _CHUNK * _FIRE
_EPAD = 811008
_NWAVES = _EPAD // _WAVE
_W4 = 132
_W1 = 66
_NPAD = 50048
_RPT = _NPAD // _NS
_EPS = 1e-5
_R = 1000
_GRID = _N // _R

_f32 = jnp.float32
_i32 = jnp.int32

_MESH = plsc.VectorSubcoreMesh(core_axis_name="c", subcore_axis_name="s")


def _edge_waves(rc_h, base, bufs, acc, gtab, npairs):
    (ibA, gbA, gsA, ssA), (ibB, gbB, gsB, ssB) = bufs

    def idx_load(w, ib):
        pltpu.sync_copy(rc_h.at[w], ib)

    def g_fire(ib, gb, sem):
        pltpu.async_copy(gtab.at[ib.at[0]], gb, sem)

    def g_wait(ib, gb, sem):
        pltpu.make_async_copy(gtab.at[ib.at[0]], gb, sem).wait()

    def s_fire(ib, gb, sem):
        pltpu.async_copy(gb, acc.at[ib.at[1]], sem, add=True)

    def s_wait(ib, gb, sem):
        pltpu.make_async_copy(gb, acc.at[ib.at[1]], sem).wait()

    idx_load(base, ibA)
    g_fire(ibA, gbA, gsA)

    def body(k, carry):
        wA = base + 2 * k
        g_wait(ibA, gbA, gsA)
        s_fire(ibA, gbA, ssA)

        @pl.when(k > 0)
        def _():
            s_wait(ibB, gbB, ssB)
        idx_load(wA + 1, ibB)
        g_fire(ibB, gbB, gsB)
        g_wait(ibB, gbB, gsB)
        s_fire(ibB, gbB, ssB)

        @pl.when(k < npairs - 1)
        def _():
            s_wait(ibA, gbA, ssA)
            idx_load(wA + 2, ibA)
            g_fire(ibA, gbA, gsA)
        return carry

    lax.fori_loop(0, npairs, body, 0)
    s_wait(ibA, gbA, ssA)
    s_wait(ibB, gbB, ssB)


def _sc_conv4_body(g0, g1, g2, g3, rc_h,
                   s0, s1, s2, s3,
                   ibA, gbA, ibB, gbB, acc, gsA, ssA, gsB, ssB):
    cid = lax.axis_index("c")
    sid = lax.axis_index("s")
    bufs = ((ibA, gbA, gsA, ssA), (ibB, gbB, gsB, ssB))
    g_refs = (g0, g1, g2, g3)
    s_refs = (s0, s1, s2, s3)
    for b in range(_NB):
        @pl.when(cid == (b % _NC))
        def _(b=b):
            g = g_refs[b]
            s = s_refs[b]
            pltpu.sync_copy(g.at[pl.ds(sid * _RPT, _RPT)],
                            acc.at[pl.ds(sid * _RPT, _RPT)])
            plsc.subcore_barrier()
            _edge_waves(rc_h, sid * _W4, bufs, acc, g, _W4 // 2)
            plsc.subcore_barrier()
            pltpu.sync_copy(acc.at[pl.ds(sid * _RPT, _RPT)],
                            s.at[pl.ds(sid * _RPT, _RPT)])
            plsc.subcore_barrier()


def _sc_conv1_body(g0, rc_h, sp,
                   ibA, gbA, ibB, gbB, acc, gsA, ssA, gsB, ssB):
    cid = lax.axis_index("c")
    sid = lax.axis_index("s")
    bufs = ((ibA, gbA, gsA, ssA), (ibB, gbB, gsB, ssB))
    pltpu.sync_copy(g0.at[pl.ds(sid * _RPT, _RPT)],
                    acc.at[pl.ds(sid * _RPT, _RPT)])
    plsc.subcore_barrier()
    _edge_waves(rc_h, (sid * _NC + cid) * _W1, bufs, acc, g0, _W1 // 2)
    plsc.subcore_barrier()
    pltpu.sync_copy(acc.at[pl.ds(sid * _RPT, _RPT)],
                    sp.at[cid, pl.ds(sid * _RPT, _RPT)])


_SC_SCRATCH = [
    pltpu.VMEM((2, _CHUNK), _i32),
    pltpu.VMEM((_CHUNK, _FB), _f32),
    pltpu.VMEM((2, _CHUNK), _i32),
    pltpu.VMEM((_CHUNK, _FB), _f32),
    pltpu.VMEM_SHARED((_NPAD, _FB), _f32),
    pltpu.SemaphoreType.DMA,
    pltpu.SemaphoreType.DMA,
    pltpu.SemaphoreType.DMA,
    pltpu.SemaphoreType.DMA,
]

_SC_PARAMS = pltpu.CompilerParams(use_tc_tiling_on_sc=False)

_sc_conv4 = pl.kernel(
    _sc_conv4_body,
    out_type=[jax.ShapeDtypeStruct((_NPAD, _FB), _f32)] * _NB,
    mesh=_MESH,
    scratch_types=_SC_SCRATCH,
    compiler_params=_SC_PARAMS,
)

_sc_conv1 = pl.kernel(
    _sc_conv1_body,
    out_type=jax.ShapeDtypeStruct((_NC, _NPAD, _FB), _f32),
    mesh=_MESH,
    scratch_types=_SC_SCRATCH,
    compiler_params=_SC_PARAMS,
)


def _dis_body(spdeg_ref, dis_ref):
    deg = spdeg_ref[0, :, 0:1] + spdeg_ref[1, :, 0:1] - 1.0
    dis_ref[...] = jnp.broadcast_to(lax.rsqrt(deg), (_R, _FB))


def _dis_call(spdeg):
    return pl.pallas_call(
        _dis_body,
        grid=(_GRID,),
        in_specs=[pl.BlockSpec((_NC, _R, _FB), lambda i: (0, i, 0))],
        out_specs=pl.BlockSpec((_R, _FB), lambda i: (i, 0)),
        out_shape=jax.ShapeDtypeStruct((_N, _FB), _f32),
    )(spdeg)


def _first_body(x_ref, dis_ref, w_ref, *g_refs):
    m = x_ref[...] * dis_ref[:, 0:x_ref.shape[1]]
    g = jnp.dot(m, w_ref[...], preferred_element_type=_f32)
    for b in range(_NB):
        g_refs[b][...] = g[:, b * _FB:(b + 1) * _FB]


def _first_call(x, dis, w):
    return pl.pallas_call(
        _first_body,
        grid=(_GRID,),
        in_specs=[
            pl.BlockSpec((_R, x.shape[1]), lambda i: (i, 0)),
            pl.BlockSpec((_R, _FB), lambda i: (i, 0)),
            pl.BlockSpec(w.shape, lambda i: (0, 0)),
        ],
        out_specs=[pl.BlockSpec((_R, _FB), lambda i: (i, 0))] * _NB,
        out_shape=[jax.ShapeDtypeStruct((_NPAD, _FB), _f32)] * _NB,
    )(x, dis, w)


def _stats_body(s0, s1, s2, s3, dis_ref, out_ref):
    @pl.when(pl.program_id(0) == 0)
    def _():
        out_ref[...] = jnp.zeros_like(out_ref)

    dis = dis_ref[...]
    su, ss = [], []
    for s in (s0, s1, s2, s3):
        c = s[...] * dis
        su.append(jnp.sum(c, axis=0))
        ss.append(jnp.sum(c * c, axis=0))
    out_ref[0, 0, :] += jnp.concatenate(su, axis=0)
    out_ref[1, 0, :] += jnp.concatenate(ss, axis=0)


def _stats_call(s_blocks, dis):
    return pl.pallas_call(
        _stats_body,
        grid=(_GRID,),
        in_specs=[pl.BlockSpec((_R, _FB), lambda i: (i, 0))] * (_NB + 1),
        out_specs=pl.BlockSpec((2, 1, _HID), lambda i: (0, 0, 0)),
        out_shape=jax.ShapeDtypeStruct((2, 1, _HID), _f32),
    )(*s_blocks, dis)


def _fuse_body(save_h, use_res, nb_out, *refs):
    refs = list(refs)
    s0, s1, s2, s3, dis_ref, sums_ref, gam_ref, bet_ref, w_ref = refs[:9]
    refs = refs[9:]
    res_ref = refs.pop(0) if use_res else None
    g_refs = refs[:nb_out]
    h_ref = refs[nb_out] if save_h else None

    s = jnp.concatenate([s0[...], s1[...], s2[...], s3[...]], axis=1)
    dis = jnp.concatenate([dis_ref[...]] * _NB, axis=1)
    c = s * dis
    mu = sums_ref[0, 0, :] * (1.0 / _N)
    var = sums_ref[1, 0, :] * (1.0 / _N) - mu * mu
    inv = lax.rsqrt(var + _EPS)
    a = gam_ref[0, :] * inv
    b0 = bet_ref[0, :] - mu * a
    h = jnp.maximum(c * a + b0, 0.0)
    if save_h:
        h_ref[...] = h
    if use_res:
        h = h + res_ref[...]
    m = h * dis
    g = jnp.dot(m, w_ref[...], preferred_element_type=_f32)
    for b in range(nb_out):
        g_refs[b][...] = g[:, b * _FB:(b + 1) * _FB]


def _fuse_call(s_blocks, dis, sums, gamma, beta, w, res=None, save_h=False):
    use_res = res is not None
    nb_out = w.shape[1] // _FB
    ins = list(s_blocks) + [dis, sums, gamma, beta, w]
    in_specs = (
        [pl.BlockSpec((_R, _FB), lambda i: (i, 0))] * _NB
        + [
            pl.BlockSpec((_R, _FB), lambda i: (i, 0)),
            pl.BlockSpec((2, 1, _HID), lambda i: (0, 0, 0)),
            pl.BlockSpec((1, _HID), lambda i: (0, 0)),
            pl.BlockSpec((1, _HID), lambda i: (0, 0)),
            pl.BlockSpec(w.shape, lambda i: (0, 0)),
        ]
    )
    if use_res:
        ins.append(res)
        in_specs.append(pl.BlockSpec((_R, _HID), lambda i: (i, 0)))
    out_specs = [pl.BlockSpec((_R, _FB), lambda i: (i, 0))] * nb_out
    out_shape = [jax.ShapeDtypeStruct((_NPAD, _FB), _f32)] * nb_out
    if save_h:
        out_specs.append(pl.BlockSpec((_R, _HID), lambda i: (i, 0)))
        out_shape.append(jax.ShapeDtypeStruct((_N, _HID), _f32))
    body = functools.partial(_fuse_body, save_h, use_res, nb_out)
    return pl.pallas_call(
        body,
        grid=(_GRID,),
        in_specs=in_specs,
        out_specs=out_specs,
        out_shape=out_shape,
    )(*ins)


def _final_body(sp_ref, g_ref, dis_ref, out_ref):
    v = (sp_ref[0, :, 0:1] + sp_ref[1, :, 0:1] - g_ref[:, 0:1]) * dis_ref[:, 0:1]
    out_ref[...] = jax.nn.sigmoid(v)


def _final_call(sp, g0, dis):
    return pl.pallas_call(
        _final_body,
        grid=(_GRID,),
        in_specs=[
            pl.BlockSpec((_NC, _R, _FB), lambda i: (0, i, 0)),
            pl.BlockSpec((_R, _FB), lambda i: (i, 0)),
            pl.BlockSpec((_R, _FB), lambda i: (i, 0)),
        ],
        out_specs=pl.BlockSpec((_R, 1), lambda i: (i, 0)),
        out_shape=jax.ShapeDtypeStruct((_N, 1), _f32),
    )(sp, g0, dis)


def kernel(x, edge_index, W_in, W_h1, W_h2, W_out,
           bn1_gamma, bn1_beta, bn2_gamma, bn2_beta):
    npadE = _EPAD - _E
    rows = jnp.concatenate(
        [edge_index[0], (jnp.arange(npadE, dtype=_i32) * 37) % _N])
    cols = jnp.concatenate(
        [edge_index[1], _N + (jnp.arange(npadE, dtype=_i32) % 16)])
    rc = jnp.stack([rows.reshape(_NWAVES, _CHUNK),
                    cols.reshape(_NWAVES, _CHUNK)], axis=1)

    ones_tab = jnp.ones((_NPAD, _FB), _f32)
    spdeg = _sc_conv1(ones_tab, rc)
    dis = _dis_call(spdeg)

    bn1_gamma = bn1_gamma.reshape(_DEPTH + 1, 1, _HID)
    bn1_beta = bn1_beta.reshape(_DEPTH + 1, 1, _HID)
    bn2_gamma = bn2_gamma.reshape(_DEPTH - 1, 1, _HID)
    bn2_beta = bn2_beta.reshape(_DEPTH - 1, 1, _HID)
    w_out_pad = jnp.pad(W_out, ((0, 0), (0, _FB - W_out.shape[1])))

    g = _first_call(x, dis, W_in)
    xs = []
    for k in range(18):
        s = _sc_conv4(*g, rc)
        sums = _stats_call(s, dis)
        if k <= 9:
            gamma, beta = bn1_gamma[k], bn1_beta[k]
        else:
            gamma, beta = bn2_gamma[k - 10], bn2_beta[k - 10]
        w_next = (W_h1[k] if k <= 8 else
                  (W_h2[k - 9] if k <= 16 else w_out_pad))
        res = xs[17 - k] if k >= 9 else None
        save_h = k <= 8
        outs = _fuse_call(s, dis, sums, gamma, beta, w_next,
                          res=res, save_h=save_h)
        if save_h:
            g, h = outs[:-1], outs[-1]
            xs.append(h)
        else:
            g = outs
        if k == 17:
            g_last = g[0]
    sp = _sc_conv1(g_last, rc)
    return _final_call(sp, g_last, dis)


# speedup vs baseline: 3.2056x; 1.0397x over previous
import functools

import jax
import jax.numpy as jnp
from jax import lax
from jax.experimental import pallas as pl
from jax.experimental.pallas import tpu as pltpu
from jax.experimental.pallas import tpu_sc as plsc

_N = 50000
_E = 800000
_HID = 128
_DEPTH = 9
_NB = 4
_FB = 32
_NS = 16
_NC = 2
_CHUNK = 384
_FIRE = 1
_WAVE = _CHUNK * _FIRE
_EPAD = 811008
_NWAVES = _EPAD // _WAVE
_W4 = 132
_W1 = 66
_NPAD = 50048
_RPT = _NPAD // _NS
_EPS = 1e-5
_R = 2000
_GRID = _N // _R

_f32 = jnp.float32
_i32 = jnp.int32

_MESH = plsc.VectorSubcoreMesh(core_axis_name="c", subcore_axis_name="s")


def _edge_waves(rc_h, base, bufs, acc, gtab, npairs):
    (ibA, gbA, gsA, ssA), (ibB, gbB, gsB, ssB) = bufs

    def idx_load(w, ib):
        pltpu.sync_copy(rc_h.at[w], ib)

    def g_fire(ib, gb, sem):
        pltpu.async_copy(gtab.at[ib.at[0]], gb, sem)

    def g_wait(ib, gb, sem):
        pltpu.make_async_copy(gtab.at[ib.at[0]], gb, sem).wait()

    def s_fire(ib, gb, sem):
        pltpu.async_copy(gb, acc.at[ib.at[1]], sem, add=True)

    def s_wait(ib, gb, sem):
        pltpu.make_async_copy(gb, acc.at[ib.at[1]], sem).wait()

    idx_load(base, ibA)
    g_fire(ibA, gbA, gsA)

    def body(k, carry):
        wA = base + 2 * k
        g_wait(ibA, gbA, gsA)
        s_fire(ibA, gbA, ssA)

        @pl.when(k > 0)
        def _():
            s_wait(ibB, gbB, ssB)
        idx_load(wA + 1, ibB)
        g_fire(ibB, gbB, gsB)
        g_wait(ibB, gbB, gsB)
        s_fire(ibB, gbB, ssB)

        @pl.when(k < npairs - 1)
        def _():
            s_wait(ibA, gbA, ssA)
            idx_load(wA + 2, ibA)
            g_fire(ibA, gbA, gsA)
        return carry

    lax.fori_loop(0, npairs, body, 0)
    s_wait(ibA, gbA, ssA)
    s_wait(ibB, gbB, ssB)


def _sc_conv4_body(g0, g1, g2, g3, rc_h,
                   s0, s1, s2, s3,
                   ibA, gbA, ibB, gbB, acc, gsA, ssA, gsB, ssB):
    cid = lax.axis_index("c")
    sid = lax.axis_index("s")
    bufs = ((ibA, gbA, gsA, ssA), (ibB, gbB, gsB, ssB))
    g_refs = (g0, g1, g2, g3)
    s_refs = (s0, s1, s2, s3)
    for b in range(_NB):
        @pl.when(cid == (b % _NC))
        def _(b=b):
            g = g_refs[b]
            s = s_refs[b]
            pltpu.sync_copy(g.at[pl.ds(sid * _RPT, _RPT)],
                            acc.at[pl.ds(sid * _RPT, _RPT)])
            plsc.subcore_barrier()
            _edge_waves(rc_h, sid * _W4, bufs, acc, g, _W4 // 2)
            plsc.subcore_barrier()
            pltpu.sync_copy(acc.at[pl.ds(sid * _RPT, _RPT)],
                            s.at[pl.ds(sid * _RPT, _RPT)])
            plsc.subcore_barrier()


def _sc_conv1_body(g0, rc_h, sp,
                   ibA, gbA, ibB, gbB, acc, gsA, ssA, gsB, ssB):
    cid = lax.axis_index("c")
    sid = lax.axis_index("s")
    bufs = ((ibA, gbA, gsA, ssA), (ibB, gbB, gsB, ssB))
    pltpu.sync_copy(g0.at[pl.ds(sid * _RPT, _RPT)],
                    acc.at[pl.ds(sid * _RPT, _RPT)])
    plsc.subcore_barrier()
    _edge_waves(rc_h, (sid * _NC + cid) * _W1, bufs, acc, g0, _W1 // 2)
    plsc.subcore_barrier()
    pltpu.sync_copy(acc.at[pl.ds(sid * _RPT, _RPT)],
                    sp.at[cid, pl.ds(sid * _RPT, _RPT)])


_SC_SCRATCH = [
    pltpu.VMEM((2, _CHUNK), _i32),
    pltpu.VMEM((_CHUNK, _FB), _f32),
    pltpu.VMEM((2, _CHUNK), _i32),
    pltpu.VMEM((_CHUNK, _FB), _f32),
    pltpu.VMEM_SHARED((_NPAD, _FB), _f32),
    pltpu.SemaphoreType.DMA,
    pltpu.SemaphoreType.DMA,
    pltpu.SemaphoreType.DMA,
    pltpu.SemaphoreType.DMA,
]

_SC_PARAMS = pltpu.CompilerParams(use_tc_tiling_on_sc=False)

_sc_conv4 = pl.kernel(
    _sc_conv4_body,
    out_type=[jax.ShapeDtypeStruct((_NPAD, _FB), _f32)] * _NB,
    mesh=_MESH,
    scratch_types=_SC_SCRATCH,
    compiler_params=_SC_PARAMS,
)

_sc_conv1 = pl.kernel(
    _sc_conv1_body,
    out_type=jax.ShapeDtypeStruct((_NC, _NPAD, _FB), _f32),
    mesh=_MESH,
    scratch_types=_SC_SCRATCH,
    compiler_params=_SC_PARAMS,
)


def _dis_body(spdeg_ref, dis_ref):
    deg = spdeg_ref[0, :, 0:1] + spdeg_ref[1, :, 0:1] - 1.0
    dis_ref[...] = jnp.broadcast_to(lax.rsqrt(deg), (_R, _FB))


def _dis_call(spdeg):
    return pl.pallas_call(
        _dis_body,
        grid=(_GRID,),
        in_specs=[pl.BlockSpec((_NC, _R, _FB), lambda i: (0, i, 0))],
        out_specs=pl.BlockSpec((_R, _FB), lambda i: (i, 0)),
        out_shape=jax.ShapeDtypeStruct((_N, _FB), _f32),
    )(spdeg)


def _first_body(x_ref, dis_ref, w_ref, *g_refs):
    m = x_ref[...] * dis_ref[:, 0:x_ref.shape[1]]
    g = jnp.dot(m, w_ref[...], preferred_element_type=_f32)
    for b in range(_NB):
        g_refs[b][...] = g[:, b * _FB:(b + 1) * _FB]


def _first_call(x, dis, w):
    return pl.pallas_call(
        _first_body,
        grid=(_GRID,),
        in_specs=[
            pl.BlockSpec((_R, x.shape[1]), lambda i: (i, 0)),
            pl.BlockSpec((_R, _FB), lambda i: (i, 0)),
            pl.BlockSpec(w.shape, lambda i: (0, 0)),
        ],
        out_specs=[pl.BlockSpec((_R, _FB), lambda i: (i, 0))] * _NB,
        out_shape=[jax.ShapeDtypeStruct((_NPAD, _FB), _f32)] * _NB,
    )(x, dis, w)


def _stats_body(s0, s1, s2, s3, dis_ref, out_ref):
    @pl.when(pl.program_id(0) == 0)
    def _():
        out_ref[...] = jnp.zeros_like(out_ref)

    dis = dis_ref[...]
    su, ss = [], []
    for s in (s0, s1, s2, s3):
        c = s[...] * dis
        su.append(jnp.sum(c, axis=0))
        ss.append(jnp.sum(c * c, axis=0))
    out_ref[0, 0, :] += jnp.concatenate(su, axis=0)
    out_ref[1, 0, :] += jnp.concatenate(ss, axis=0)


def _stats_call(s_blocks, dis):
    return pl.pallas_call(
        _stats_body,
        grid=(_GRID,),
        in_specs=[pl.BlockSpec((_R, _FB), lambda i: (i, 0))] * (_NB + 1),
        out_specs=pl.BlockSpec((2, 1, _HID), lambda i: (0, 0, 0)),
        out_shape=jax.ShapeDtypeStruct((2, 1, _HID), _f32),
    )(*s_blocks, dis)


def _fuse_body(save_h, use_res, nb_out, *refs):
    refs = list(refs)
    s0, s1, s2, s3, dis_ref, sums_ref, gam_ref, bet_ref, w_ref = refs[:9]
    refs = refs[9:]
    res_ref = refs.pop(0) if use_res else None
    g_refs = refs[:nb_out]
    h_ref = refs[nb_out] if save_h else None

    s = jnp.concatenate([s0[...], s1[...], s2[...], s3[...]], axis=1)
    dis = jnp.concatenate([dis_ref[...]] * _NB, axis=1)
    c = s * dis
    mu = sums_ref[0, 0, :] * (1.0 / _N)
    var = sums_ref[1, 0, :] * (1.0 / _N) - mu * mu
    inv = lax.rsqrt(var + _EPS)
    a = gam_ref[0, :] * inv
    b0 = bet_ref[0, :] - mu * a
    h = jnp.maximum(c * a + b0, 0.0)
    if save_h:
        h_ref[...] = h
    if use_res:
        h = h + res_ref[...]
    m = h * dis
    g = jnp.dot(m, w_ref[...], preferred_element_type=_f32)
    for b in range(nb_out):
        g_refs[b][...] = g[:, b * _FB:(b + 1) * _FB]


def _fuse_call(s_blocks, dis, sums, gamma, beta, w, res=None, save_h=False):
    use_res = res is not None
    nb_out = w.shape[1] // _FB
    ins = list(s_blocks) + [dis, sums, gamma, beta, w]
    in_specs = (
        [pl.BlockSpec((_R, _FB), lambda i: (i, 0))] * _NB
        + [
            pl.BlockSpec((_R, _FB), lambda i: (i, 0)),
            pl.BlockSpec((2, 1, _HID), lambda i: (0, 0, 0)),
            pl.BlockSpec((1, _HID), lambda i: (0, 0)),
            pl.BlockSpec((1, _HID), lambda i: (0, 0)),
            pl.BlockSpec(w.shape, lambda i: (0, 0)),
        ]
    )
    if use_res:
        ins.append(res)
        in_specs.append(pl.BlockSpec((_R, _HID), lambda i: (i, 0)))
    out_specs = [pl.BlockSpec((_R, _FB), lambda i: (i, 0))] * nb_out
    out_shape = [jax.ShapeDtypeStruct((_NPAD, _FB), _f32)] * nb_out
    if save_h:
        out_specs.append(pl.BlockSpec((_R, _HID), lambda i: (i, 0)))
        out_shape.append(jax.ShapeDtypeStruct((_N, _HID), _f32))
    body = functools.partial(_fuse_body, save_h, use_res, nb_out)
    return pl.pallas_call(
        body,
        grid=(_GRID,),
        in_specs=in_specs,
        out_specs=out_specs,
        out_shape=out_shape,
    )(*ins)


def _final_body(sp_ref, g_ref, dis_ref, out_ref):
    v = (sp_ref[0, :, 0:1] + sp_ref[1, :, 0:1] - g_ref[:, 0:1]) * dis_ref[:, 0:1]
    out_ref[...] = jax.nn.sigmoid(v)


def _final_call(sp, g0, dis):
    return pl.pallas_call(
        _final_body,
        grid=(_GRID,),
        in_specs=[
            pl.BlockSpec((_NC, _R, _FB), lambda i: (0, i, 0)),
            pl.BlockSpec((_R, _FB), lambda i: (i, 0)),
            pl.BlockSpec((_R, _FB), lambda i: (i, 0)),
        ],
        out_specs=pl.BlockSpec((_R, 1), lambda i: (i, 0)),
        out_shape=jax.ShapeDtypeStruct((_N, 1), _f32),
    )(sp, g0, dis)


def kernel(x, edge_index, W_in, W_h1, W_h2, W_out,
           bn1_gamma, bn1_beta, bn2_gamma, bn2_beta):
    npadE = _EPAD - _E
    rows = jnp.concatenate(
        [edge_index[0], (jnp.arange(npadE, dtype=_i32) * 37) % _N])
    cols = jnp.concatenate(
        [edge_index[1], _N + (jnp.arange(npadE, dtype=_i32) % 16)])
    rc = jnp.stack([rows.reshape(_NWAVES, _CHUNK),
                    cols.reshape(_NWAVES, _CHUNK)], axis=1)

    ones_tab = jnp.ones((_NPAD, _FB), _f32)
    spdeg = _sc_conv1(ones_tab, rc)
    dis = _dis_call(spdeg)

    bn1_gamma = bn1_gamma.reshape(_DEPTH + 1, 1, _HID)
    bn1_beta = bn1_beta.reshape(_DEPTH + 1, 1, _HID)
    bn2_gamma = bn2_gamma.reshape(_DEPTH - 1, 1, _HID)
    bn2_beta = bn2_beta.reshape(_DEPTH - 1, 1, _HID)
    w_out_pad = jnp.pad(W_out, ((0, 0), (0, _FB - W_out.shape[1])))

    g = _first_call(x, dis, W_in)
    xs = []
    for k in range(18):
        s = _sc_conv4(*g, rc)
        sums = _stats_call(s, dis)
        if k <= 9:
            gamma, beta = bn1_gamma[k], bn1_beta[k]
        else:
            gamma, beta = bn2_gamma[k - 10], bn2_beta[k - 10]
        w_next = (W_h1[k] if k <= 8 else
                  (W_h2[k - 9] if k <= 16 else w_out_pad))
        res = xs[17 - k] if k >= 9 else None
        save_h = k <= 8
        outs = _fuse_call(s, dis, sums, gamma, beta, w_next,
                          res=res, save_h=save_h)
        if save_h:
            g, h = outs[:-1], outs[-1]
            xs.append(h)
        else:
            g = outs
        if k == 17:
            g_last = g[0]
    sp = _sc_conv1(g_last, rc)
    return _final_call(sp, g_last, dis)
